# baseline (speedup 1.0000x reference)
import functools

import jax
import jax.numpy as jnp
from jax import lax
from jax.experimental import pallas as pl
from jax.experimental.pallas import tpu as pltpu
from jax.experimental.pallas import tpu_sc as plsc

N = 10000
E = 320000
D = 128
ROWS = 1000

NC = 2
NS = 16
NW = NC * NS
NP = 10240
EPW = E // NW
DEG_CHUNK = 2000


def _deg_body(dst_hbm, out_hbm, dstv0, dstv1, hist, semd0, semd1):
    wid = lax.axis_index("s") * NC + lax.axis_index("c")
    zeros = jnp.zeros((16,), jnp.float32)
    ones = jnp.ones((16,), jnp.float32)
    dstv = (dstv0, dstv1)
    semd = (semd0, semd1)

    base = wid * EPW
    nchunks = EPW // DEG_CHUNK
    pltpu.async_copy(dst_hbm.at[pl.ds(base, DEG_CHUNK)], dstv0, semd0)

    def zero_body(i, _):
        hist[pl.ds(i * 16, 16)] = zeros
        return 0

    lax.fori_loop(0, NP // 16, zero_body, 0)

    for c in range(nchunks):
        k = c % 2
        pltpu.make_async_copy(dst_hbm.at[pl.ds(base, DEG_CHUNK)], dstv[k],
                              semd[k]).wait()
        if c + 1 < nchunks:
            pltpu.async_copy(
                dst_hbm.at[pl.ds(base + (c + 1) * DEG_CHUNK, DEG_CHUNK)],
                dstv[(c + 1) % 2], semd[(c + 1) % 2])

        def scat_body(j, _):
            idx = dstv[k][pl.ds(j * 16, 16)]
            plsc.addupdate_scatter(hist, [idx], ones)
            return 0

        lax.fori_loop(0, DEG_CHUNK // 16, scat_body, 0)
    pltpu.sync_copy(hist, out_hbm.at[wid])


_deg_kernel = functools.partial(
    pl.kernel,
    out_type=jax.ShapeDtypeStruct((NW, NP), jnp.float32),
    mesh=plsc.VectorSubcoreMesh(core_axis_name="c", subcore_axis_name="s"),
    scratch_types=[
        pltpu.VMEM((DEG_CHUNK,), jnp.int32),
        pltpu.VMEM((DEG_CHUNK,), jnp.int32),
        pltpu.VMEM((NP,), jnp.float32),
        pltpu.SemaphoreType.DMA,
        pltpu.SemaphoreType.DMA,
    ],
    compiler_params=pltpu.CompilerParams(needs_layout_passes=False),
)(_deg_body)


EK = 80
ECHUNKS = EPW // EK
ZROWS = 128
SPN = NP // NS


RB = 3
IRB = 6


def _edge_body(hh_hbm, src_hbm, dst_hbm, out_hbm, refs, agg, semas):
    cid = lax.axis_index("c")
    sid = lax.axis_index("s")
    wid = sid * NC + cid
    sidx = refs[:IRB]
    didx = refs[IRB:2 * IRB]
    rows = refs[2 * IRB:]
    semg = semas[:RB]
    sems = semas[RB:2 * RB]
    semi = semas[2 * RB:]
    zeros = jnp.zeros((16,), jnp.float32)
    ebase = wid * EPW

    r0 = sid * SPN

    @pl.when(cid == 0)
    def _():
        nrows = jnp.minimum(SPN, jnp.maximum(0, N - r0))
        for t in range(SPN // EK):
            @pl.when(t * EK + EK <= nrows)
            def _():
                pltpu.sync_copy(hh_hbm.at[pl.ds(r0 + t * EK, EK)],
                                agg.at[pl.ds(r0 + t * EK, EK)])

    @pl.when(cid == 1)
    def _():
        def zb_body(i, _):
            for j in range(D // 16):
                rows[0][i, pl.ds(j * 16, 16)] = zeros
            return 0

        lax.fori_loop(0, EK, zb_body, 0)
        for t in range(SPN // EK):
            pltpu.sync_copy(rows[0], agg.at[pl.ds(r0 + t * EK, EK)])

    plsc.subcore_barrier()

    def idx_load(c, k8, sem):
        off = ebase + c * EK
        pltpu.async_copy(src_hbm.at[pl.ds(off, EK)], sidx[k8], sem)
        pltpu.async_copy(dst_hbm.at[pl.ds(off, EK)], didx[k8], sem)

    def idx_wait(k8):
        pltpu.make_async_copy(src_hbm.at[pl.ds(ebase, EK)], sidx[k8],
                              semi[k8]).wait()
        pltpu.make_async_copy(dst_hbm.at[pl.ds(ebase, EK)], didx[k8],
                              semi[k8]).wait()

    pltpu.sync_copy(src_hbm.at[pl.ds(ebase, EK)], sidx[0])
    pltpu.sync_copy(dst_hbm.at[pl.ds(ebase, EK)], didx[0])
    pltpu.sync_copy(src_hbm.at[pl.ds(ebase + EK, EK)], sidx[1])
    pltpu.sync_copy(dst_hbm.at[pl.ds(ebase + EK, EK)], didx[1])
    idx_load(2, 2, semi[2])
    idx_load(3, 3, semi[3])
    pltpu.async_copy(hh_hbm.at[sidx[0]], rows[0], semg[0])
    pltpu.async_copy(hh_hbm.at[sidx[1]], rows[1], semg[1])

    def chunk_body(c, _):
        def step(k8):
            k = k8 % RB
            k2 = (k8 + 2) % RB
            k82 = (k8 + 2) % IRB
            k84 = (k8 + 4) % IRB
            pltpu.make_async_copy(hh_hbm.at[sidx[k8]], rows[k],
                                  semg[k]).wait()
            pltpu.async_copy(rows[k], agg.at[didx[k8]], sems[k], add=True)

            @pl.when(c >= RB - 2)
            def _():
                pltpu.make_async_copy(rows[k2], agg.at[didx[k8]],
                                      sems[k2]).wait()

            @pl.when(c + 2 < ECHUNKS)
            def _():
                idx_wait(k82)
                pltpu.async_copy(hh_hbm.at[sidx[k82]], rows[k2], semg[k2])

            @pl.when(c + 4 < ECHUNKS)
            def _():
                idx_load(c + 4, k84, semi[k84])

        for k8 in range(IRB):
            @pl.when(c % IRB == k8)
            def _(k8=k8):
                step(k8)

        return 0

    lax.fori_loop(0, ECHUNKS, chunk_body, 0)
    for j in range(RB - 2):
        pltpu.make_async_copy(rows[(ECHUNKS - 1 - j) % RB], agg.at[didx[0]],
                              sems[(ECHUNKS - 1 - j) % RB]).wait()
    plsc.subcore_barrier()
    for t in range(SPN // EK):
        pltpu.sync_copy(agg.at[pl.ds(r0 + t * EK, EK)],
                        out_hbm.at[cid, pl.ds(r0 + t * EK, EK)])


_edge_kernel = functools.partial(
    pl.kernel,
    out_type=jax.ShapeDtypeStruct((NC, NP, D), jnp.float32),
    mesh=plsc.VectorSubcoreMesh(core_axis_name="c", subcore_axis_name="s"),
    scratch_types=[
        [pltpu.VMEM((EK,), jnp.int32)] * (2 * IRB)
        + [pltpu.VMEM((EK, D), jnp.float32)] * RB,
        pltpu.VMEM_SHARED((NP, D), jnp.float32),
        [pltpu.SemaphoreType.DMA] * (2 * RB + IRB),
    ],
    compiler_params=pltpu.CompilerParams(needs_layout_passes=False),
)(_edge_body)


def _pre_body(x_ref, w1_ref, b1_ref, W_ref, dinv_ref, hh_ref):
    xb = x_ref[...]
    mean = jnp.mean(xb, axis=1, keepdims=True)
    cen = xb - mean
    var = jnp.mean(cen * cen, axis=1, keepdims=True)
    h1 = cen * jax.lax.rsqrt(var + 1e-5) * w1_ref[...] + b1_ref[...]
    h = jnp.dot(h1, W_ref[...], preferred_element_type=jnp.float32)
    hh_ref[...] = h * dinv_ref[...]


def _tc_pre(x, ln1_w, ln1_b, gcn_W, dinv_col):
    grid = (N // ROWS,)
    return pl.pallas_call(
        _pre_body,
        grid=grid,
        in_specs=[
            pl.BlockSpec((ROWS, D), lambda i: (i, 0)),
            pl.BlockSpec((D,), lambda i: (0,)),
            pl.BlockSpec((D,), lambda i: (0,)),
            pl.BlockSpec((D, D), lambda i: (0, 0)),
            pl.BlockSpec((ROWS, 1), lambda i: (i, 0)),
        ],
        out_specs=pl.BlockSpec((ROWS, D), lambda i: (i, 0)),
        out_shape=jax.ShapeDtypeStruct((N, D), jnp.float32),
    )(x, ln1_w, ln1_b, gcn_W, dinv_col)


def _post_body(x_ref, a0_ref, a1_ref, dinv_ref, gb_ref, s1_ref,
               w2_ref, b2_ref, ffW_ref, ffb_ref, s2_ref, out_ref):
    agg = a0_ref[0] + a1_ref[0]
    m = agg * dinv_ref[...] + gb_ref[...]
    m = m * jax.nn.sigmoid(m)
    x1 = x_ref[...] + s1_ref[0, 0] * m
    mean = jnp.mean(x1, axis=1, keepdims=True)
    cen = x1 - mean
    var = jnp.mean(cen * cen, axis=1, keepdims=True)
    h2 = cen * jax.lax.rsqrt(var + 1e-5) * w2_ref[...] + b2_ref[...]
    f = jnp.dot(h2, ffW_ref[...], preferred_element_type=jnp.float32) + ffb_ref[...]
    f = f * jax.nn.sigmoid(f)
    out_ref[...] = x1 + s2_ref[0, 0] * f


def _tc_post(x, aggs, dinv_col, gcn_b, alpha1, ln2_w, ln2_b,
             ff_W, ff_b, alpha2):
    grid = (N // ROWS,)
    row = lambda i: (i, 0)
    vec = lambda i: (0,)
    smem = pl.BlockSpec(memory_space=pltpu.SMEM)
    return pl.pallas_call(
        _post_body,
        grid=grid,
        in_specs=[
            pl.BlockSpec((ROWS, D), row),
            pl.BlockSpec((1, ROWS, D), lambda i: (0, i, 0)),
            pl.BlockSpec((1, ROWS, D), lambda i: (1, i, 0)),
            pl.BlockSpec((ROWS, 1), row),
            pl.BlockSpec((D,), vec),
            smem,
            pl.BlockSpec((D,), vec),
            pl.BlockSpec((D,), vec),
            pl.BlockSpec((D, D), lambda i: (0, 0)),
            pl.BlockSpec((D,), vec),
            smem,
        ],
        out_specs=pl.BlockSpec((ROWS, D), row),
        out_shape=jax.ShapeDtypeStruct((N, D), jnp.float32),
    )(x, aggs, aggs, dinv_col, gcn_b, alpha1.reshape(1, 1), ln2_w,
      ln2_b, ff_W, ff_b, alpha2.reshape(1, 1))


def kernel(x, edge_index, ln1_w, ln1_b, gcn_W, gcn_b, alpha1, alpha2,
           ln2_w, ln2_b, ff_W, ff_b):
    src = edge_index[0]
    dst = edge_index[1]
    deg_parts = _deg_kernel(dst)
    deg = deg_parts.sum(axis=0)[:N] + 1.0
    dinv_col = jax.lax.rsqrt(deg).reshape(N, 1)
    hh = _tc_pre(x, ln1_w, ln1_b, gcn_W, dinv_col)
    aggs = _edge_kernel(hh, src, dst)
    return _tc_post(x, aggs, dinv_col, gcn_b, alpha1, ln2_w,
                    ln2_b, ff_W, ff_b, alpha2)


# speedup vs baseline: 1.0053x; 1.0053x over previous
import functools

import jax
import jax.numpy as jnp
from jax import lax
from jax.experimental import pallas as pl
from jax.experimental.pallas import tpu as pltpu
from jax.experimental.pallas import tpu_sc as plsc

N = 10000
E = 320000
D = 128
ROWS = 1000

NC = 2
NS = 16
NW = NC * NS
NP = 10240
EPW = E // NW
DEG_CHUNK = 2000


def _deg_body(dst_hbm, out_hbm, dstv, hist):
    wid = lax.axis_index("s") * NC + lax.axis_index("c")
    zeros = jnp.zeros((16,), jnp.float32)
    ones = jnp.ones((16,), jnp.float32)

    def zero_body(i, _):
        hist[pl.ds(i * 16, 16)] = zeros
        return 0

    lax.fori_loop(0, NP // 16, zero_body, 0)

    base = wid * EPW
    for c in range(EPW // DEG_CHUNK):
        pltpu.sync_copy(dst_hbm.at[pl.ds(base + c * DEG_CHUNK, DEG_CHUNK)], dstv)

        def scat_body(j, _):
            idx = dstv[pl.ds(j * 16, 16)]
            plsc.addupdate_scatter(hist, [idx], ones)
            return 0

        lax.fori_loop(0, DEG_CHUNK // 16, scat_body, 0)
    pltpu.sync_copy(hist, out_hbm.at[wid])


_deg_kernel = functools.partial(
    pl.kernel,
    out_type=jax.ShapeDtypeStruct((NW, NP), jnp.float32),
    mesh=plsc.VectorSubcoreMesh(core_axis_name="c", subcore_axis_name="s"),
    scratch_types=[
        pltpu.VMEM((DEG_CHUNK,), jnp.int32),
        pltpu.VMEM((NP,), jnp.float32),
    ],
    compiler_params=pltpu.CompilerParams(needs_layout_passes=False),
)(_deg_body)


EK = 80
ECHUNKS = EPW // EK
ZROWS = 128
SPN = NP // NS


RB = 3
IRB = 6


def _edge_body(hh_hbm, src_hbm, dst_hbm, out_hbm, refs, agg, semas):
    cid = lax.axis_index("c")
    sid = lax.axis_index("s")
    wid = sid * NC + cid
    sidx = refs[:IRB]
    didx = refs[IRB:2 * IRB]
    rows = refs[2 * IRB:]
    semg = semas[:RB]
    sems = semas[RB:2 * RB]
    semi = semas[2 * RB:]
    zeros = jnp.zeros((16,), jnp.float32)
    ebase = wid * EPW

    def zb_body(i, _):
        for j in range(D // 16):
            rows[0][i, pl.ds(j * 16, 16)] = zeros
        return 0

    lax.fori_loop(0, EK, zb_body, 0)
    r0 = sid * SPN
    for t in range(SPN // EK):
        pltpu.sync_copy(rows[0], agg.at[pl.ds(r0 + t * EK, EK)])
    plsc.subcore_barrier()

    def idx_load(c, k8, sem):
        off = ebase + c * EK
        pltpu.async_copy(src_hbm.at[pl.ds(off, EK)], sidx[k8], sem)
        pltpu.async_copy(dst_hbm.at[pl.ds(off, EK)], didx[k8], sem)

    def idx_wait(k8):
        pltpu.make_async_copy(src_hbm.at[pl.ds(ebase, EK)], sidx[k8],
                              semi[k8]).wait()
        pltpu.make_async_copy(dst_hbm.at[pl.ds(ebase, EK)], didx[k8],
                              semi[k8]).wait()

    pltpu.sync_copy(src_hbm.at[pl.ds(ebase, EK)], sidx[0])
    pltpu.sync_copy(dst_hbm.at[pl.ds(ebase, EK)], didx[0])
    pltpu.sync_copy(src_hbm.at[pl.ds(ebase + EK, EK)], sidx[1])
    pltpu.sync_copy(dst_hbm.at[pl.ds(ebase + EK, EK)], didx[1])
    idx_load(2, 2, semi[2])
    idx_load(3, 3, semi[3])
    pltpu.async_copy(hh_hbm.at[sidx[0]], rows[0], semg[0])
    pltpu.async_copy(hh_hbm.at[sidx[1]], rows[1], semg[1])

    def chunk_body(c, _):
        def step(k8):
            k = k8 % RB
            k2 = (k8 + 2) % RB
            k82 = (k8 + 2) % IRB
            k84 = (k8 + 4) % IRB
            pltpu.make_async_copy(hh_hbm.at[sidx[k8]], rows[k],
                                  semg[k]).wait()
            pltpu.async_copy(rows[k], agg.at[didx[k8]], sems[k], add=True)

            @pl.when(c >= RB - 2)
            def _():
                pltpu.make_async_copy(rows[k2], agg.at[didx[k8]],
                                      sems[k2]).wait()

            @pl.when(c + 2 < ECHUNKS)
            def _():
                idx_wait(k82)
                pltpu.async_copy(hh_hbm.at[sidx[k82]], rows[k2], semg[k2])

            @pl.when(c + 4 < ECHUNKS)
            def _():
                idx_load(c + 4, k84, semi[k84])

        for k8 in range(IRB):
            @pl.when(c % IRB == k8)
            def _(k8=k8):
                step(k8)

        return 0

    lax.fori_loop(0, ECHUNKS, chunk_body, 0)
    for j in range(RB - 2):
        pltpu.make_async_copy(rows[(ECHUNKS - 1 - j) % RB], agg.at[didx[0]],
                              sems[(ECHUNKS - 1 - j) % RB]).wait()
    plsc.subcore_barrier()
    for t in range(SPN // EK):
        pltpu.sync_copy(agg.at[pl.ds(r0 + t * EK, EK)],
                        out_hbm.at[cid, pl.ds(r0 + t * EK, EK)])


_edge_kernel = functools.partial(
    pl.kernel,
    out_type=jax.ShapeDtypeStruct((NC, NP, D), jnp.float32),
    mesh=plsc.VectorSubcoreMesh(core_axis_name="c", subcore_axis_name="s"),
    scratch_types=[
        [pltpu.VMEM((EK,), jnp.int32)] * (2 * IRB)
        + [pltpu.VMEM((EK, D), jnp.float32)] * RB,
        pltpu.VMEM_SHARED((NP, D), jnp.float32),
        [pltpu.SemaphoreType.DMA] * (2 * RB + IRB),
    ],
    compiler_params=pltpu.CompilerParams(needs_layout_passes=False),
)(_edge_body)


def _pre_body(x_ref, w1_ref, b1_ref, W_ref, dinv_ref, hh_ref):
    xb = x_ref[...]
    mean = jnp.mean(xb, axis=1, keepdims=True)
    cen = xb - mean
    var = jnp.mean(cen * cen, axis=1, keepdims=True)
    h1 = cen * jax.lax.rsqrt(var + 1e-5) * w1_ref[...] + b1_ref[...]
    h = jnp.dot(h1, W_ref[...], preferred_element_type=jnp.float32)
    hh_ref[...] = h * dinv_ref[...]


def _tc_pre(x, ln1_w, ln1_b, gcn_W, dinv_col):
    grid = (N // ROWS,)
    return pl.pallas_call(
        _pre_body,
        grid=grid,
        in_specs=[
            pl.BlockSpec((ROWS, D), lambda i: (i, 0)),
            pl.BlockSpec((D,), lambda i: (0,)),
            pl.BlockSpec((D,), lambda i: (0,)),
            pl.BlockSpec((D, D), lambda i: (0, 0)),
            pl.BlockSpec((ROWS, 1), lambda i: (i, 0)),
        ],
        out_specs=pl.BlockSpec((ROWS, D), lambda i: (i, 0)),
        out_shape=jax.ShapeDtypeStruct((N, D), jnp.float32),
    )(x, ln1_w, ln1_b, gcn_W, dinv_col)


def _post_body(x_ref, hh_ref, a0_ref, a1_ref, dinv_ref, gb_ref, s1_ref,
               w2_ref, b2_ref, ffW_ref, ffb_ref, s2_ref, out_ref):
    agg = a0_ref[0] + a1_ref[0] + hh_ref[...]
    m = agg * dinv_ref[...] + gb_ref[...]
    m = m * jax.nn.sigmoid(m)
    x1 = x_ref[...] + s1_ref[0, 0] * m
    mean = jnp.mean(x1, axis=1, keepdims=True)
    cen = x1 - mean
    var = jnp.mean(cen * cen, axis=1, keepdims=True)
    h2 = cen * jax.lax.rsqrt(var + 1e-5) * w2_ref[...] + b2_ref[...]
    f = jnp.dot(h2, ffW_ref[...], preferred_element_type=jnp.float32) + ffb_ref[...]
    f = f * jax.nn.sigmoid(f)
    out_ref[...] = x1 + s2_ref[0, 0] * f


def _tc_post(x, hh, aggs, dinv_col, gcn_b, alpha1, ln2_w, ln2_b,
             ff_W, ff_b, alpha2):
    grid = (N // ROWS,)
    row = lambda i: (i, 0)
    vec = lambda i: (0,)
    smem = pl.BlockSpec(memory_space=pltpu.SMEM)
    return pl.pallas_call(
        _post_body,
        grid=grid,
        in_specs=[
            pl.BlockSpec((ROWS, D), row),
            pl.BlockSpec((ROWS, D), row),
            pl.BlockSpec((1, ROWS, D), lambda i: (0, i, 0)),
            pl.BlockSpec((1, ROWS, D), lambda i: (1, i, 0)),
            pl.BlockSpec((ROWS, 1), row),
            pl.BlockSpec((D,), vec),
            smem,
            pl.BlockSpec((D,), vec),
            pl.BlockSpec((D,), vec),
            pl.BlockSpec((D, D), lambda i: (0, 0)),
            pl.BlockSpec((D,), vec),
            smem,
        ],
        out_specs=pl.BlockSpec((ROWS, D), row),
        out_shape=jax.ShapeDtypeStruct((N, D), jnp.float32),
    )(x, hh, aggs, aggs, dinv_col, gcn_b, alpha1.reshape(1, 1), ln2_w,
      ln2_b, ff_W, ff_b, alpha2.reshape(1, 1))


def kernel(x, edge_index, ln1_w, ln1_b, gcn_W, gcn_b, alpha1, alpha2,
           ln2_w, ln2_b, ff_W, ff_b):
    src = edge_index[0]
    dst = edge_index[1]
    deg_parts = _deg_kernel(dst)
    deg = deg_parts.sum(axis=0)[:N] + 1.0
    dinv_col = jax.lax.rsqrt(deg).reshape(N, 1)
    hh = _tc_pre(x, ln1_w, ln1_b, gcn_W, dinv_col)
    aggs = _edge_kernel(hh, src, dst)
    return _tc_post(x, hh, aggs, dinv_col, gcn_b, alpha1, ln2_w,
                    ln2_b, ff_W, ff_b, alpha2)
